# Initial kernel scaffold; baseline (speedup 1.0000x reference)
import jax
import jax.numpy as jnp
from jax.experimental import pallas as pl


def kernel(x, edge_index, edge_attr, batch, en1_w1, en1_b1, en1_w2, en1_b2, root1, bias1, en2_w1, en2_b1, en2_w2, en2_b2, root2, bias2, mlp_w1, mlp_b1, mlp_w2, mlp_b2):
    raise NotImplementedError("write your pallas kernel here")


# speedup vs baseline: 1.6410x; 1.6410x over previous
import functools

import jax
import jax.numpy as jnp
from jax import lax
from jax.experimental import pallas as pl
from jax.experimental.pallas import tpu as pltpu
from jax.experimental.pallas import tpu_sc as plsc

_N = 10000
_E = 160000
_IN = 128
_ED = 16
_H = 8
_G = 64

_NWORK = 32
_CH = 128
_CHUNKS = 40
_EP = _NWORK * _CHUNKS * _CH
_NP = 10240
_W = 128


def _make_gather(D):
    mesh = plsc.VectorSubcoreMesh(core_axis_name="c", subcore_axis_name="s")
    per_w = _EP // _NWORK

    @functools.partial(
        pl.kernel, mesh=mesh,
        out_type=jax.ShapeDtypeStruct((_EP, D), jnp.float32),
        scratch_types=[
            pltpu.VMEM((_CH,), jnp.int32),
            pltpu.VMEM((_CH, D), jnp.float32),
            pltpu.SemaphoreType.DMA,
        ],
    )
    def gather_k(table_hbm, idx_hbm, out_hbm, idx_v, rows_v, sem):
        wid = lax.axis_index("c") * 16 + lax.axis_index("s")
        base = wid * per_w

        def body(j, carry):
            off = base + j * _CH
            pltpu.sync_copy(idx_hbm.at[pl.ds(off, _CH)], idx_v)
            pltpu.async_copy(table_hbm.at[idx_v], rows_v, sem).wait()
            pltpu.sync_copy(rows_v, out_hbm.at[pl.ds(off, _CH)])
            return carry

        lax.fori_loop(0, _CHUNKS, body, 0)

    return gather_k


def _make_scatter():
    mesh = plsc.VectorSubcoreMesh(core_axis_name="c", subcore_axis_name="s")
    per_w = _EP // _NWORK
    rpt = _NP // 16

    @functools.partial(
        pl.kernel, mesh=mesh,
        out_type=jax.ShapeDtypeStruct((2, _NP, _W), jnp.float32),
        scratch_types=[
            pltpu.VMEM((_CH,), jnp.int32),
            pltpu.VMEM((_CH, _W), jnp.float32),
            pltpu.VMEM_SHARED((_NP, _W), jnp.float32),
            pltpu.SemaphoreType.DMA,
        ],
    )
    def scatter_k(msg_hbm, dst_hbm, zeros_hbm, out_hbm, idx_v, rows_v, acc_sh,
                  sem):
        cid = lax.axis_index("c")
        sid = lax.axis_index("s")
        wid = cid * 16 + sid

        pltpu.sync_copy(zeros_hbm, acc_sh.at[pl.ds(sid * rpt, rpt)])
        plsc.subcore_barrier()

        base = wid * per_w

        def body(j, carry):
            off = base + j * _CH
            pltpu.sync_copy(dst_hbm.at[pl.ds(off, _CH)], idx_v)
            pltpu.sync_copy(msg_hbm.at[pl.ds(off, _CH)], rows_v)
            pltpu.sync_copy(rows_v, acc_sh.at[idx_v], add=True)
            return carry

        lax.fori_loop(0, _CHUNKS, body, 0)
        plsc.subcore_barrier()

        pltpu.sync_copy(acc_sh.at[pl.ds(sid * rpt, rpt)],
                        out_hbm.at[cid, pl.ds(sid * rpt, rpt)])

    return scatter_k


def _edge_body(ea_ref, xs_ref, w1_ref, b1_ref, w2p_ref, b2p_ref, out_ref, *,
               in_dim):
    h = jnp.maximum(
        jnp.dot(ea_ref[...], w1_ref[...], preferred_element_type=jnp.float32)
        + b1_ref[...], 0.0)
    wf = jnp.dot(h, w2p_ref[...], preferred_element_type=jnp.float32) + b2p_ref[...]
    xs = xs_ref[...][:, :in_dim]
    xs8 = jnp.concatenate([xs] * 8, axis=1)
    t = xs8 * wf
    ko = wf.shape[1]
    gsz = ko // 8
    cc = lax.broadcasted_iota(jnp.int32, (ko, 8), 0) // gsz
    oo = lax.broadcasted_iota(jnp.int32, (ko, 8), 1)
    sel = (cc == oo).astype(jnp.float32)
    msg = jnp.dot(t, sel, preferred_element_type=jnp.float32)
    b = msg.shape[0]
    out_ref[...] = jnp.concatenate(
        [msg, jnp.ones((b, 1), jnp.float32),
         jnp.zeros((b, _W - 9), jnp.float32)], axis=1)


def _edge_call(ea, xs, w1, b1, w2p, b2p, block, in_dim):
    ep = ea.shape[0]
    return pl.pallas_call(
        functools.partial(_edge_body, in_dim=in_dim),
        grid=(ep // block,),
        in_specs=[
            pl.BlockSpec((block, ea.shape[1]), lambda i: (i, 0)),
            pl.BlockSpec((block, xs.shape[1]), lambda i: (i, 0)),
            pl.BlockSpec(w1.shape, lambda i: (0, 0)),
            pl.BlockSpec(b1.shape, lambda i: (0, 0)),
            pl.BlockSpec(w2p.shape, lambda i: (0, 0)),
            pl.BlockSpec(b2p.shape, lambda i: (0, 0)),
        ],
        out_specs=pl.BlockSpec((block, _W), lambda i: (i, 0)),
        out_shape=jax.ShapeDtypeStruct((ep, _W), jnp.float32),
    )(ea, xs, w1, b1, w2p, b2p)


def _node_body(x_ref, p_ref, root_ref, bias_ref, out_ref):
    p = p_ref[0] + p_ref[1]
    s = p[:_N, :8]
    cnt = p[:_N, 8:9]
    mean = s / jnp.maximum(cnt, 1.0)
    hn = jnp.maximum(
        jnp.dot(x_ref[...], root_ref[...], preferred_element_type=jnp.float32)
        + mean + bias_ref[...], 0.0)
    out_ref[...] = jnp.concatenate(
        [hn, jnp.zeros((_N, _IN - _H), jnp.float32)], axis=1)


def _node_call(x, p, root, bias):
    return pl.pallas_call(
        _node_body,
        out_shape=jax.ShapeDtypeStruct((_N, _IN), jnp.float32),
    )(x, p, root, bias)


def _final_body(hn_ref, p_ref, batch_ref, root_ref, bias_ref,
                w1_ref, b1_ref, w2_ref, b2_ref, out_ref):
    p = p_ref[0] + p_ref[1]
    s = p[:_N, :8]
    cnt = p[:_N, 8:9]
    mean = s / jnp.maximum(cnt, 1.0)
    h2 = jnp.maximum(
        jnp.dot(hn_ref[...][:, :_H], root_ref[...],
                preferred_element_type=jnp.float32)
        + mean + bias_ref[...], 0.0)
    bvec = batch_ref[...]
    mask = (jnp.broadcast_to(bvec, (_G, _N))
            == lax.broadcasted_iota(jnp.int32, (_G, _N), 0)).astype(jnp.float32)
    gs = jnp.dot(mask, h2, preferred_element_type=jnp.float32)
    gc = jnp.sum(mask, axis=1, keepdims=True)
    g = gs / jnp.maximum(gc, 1.0)
    z = jnp.maximum(
        jnp.dot(g, w1_ref[...], preferred_element_type=jnp.float32)
        + b1_ref[...], 0.0)
    out_ref[...] = jnp.dot(z, w2_ref[...], preferred_element_type=jnp.float32) + b2_ref[...]


def _final_call(hn, p, batch2d, root, bias, w1, b1, w2, b2):
    return pl.pallas_call(
        _final_body,
        out_shape=jax.ShapeDtypeStruct((_G, 1), jnp.float32),
    )(hn, p, batch2d, root, bias, w1, b1, w2, b2)


_make_gather = functools.lru_cache(maxsize=None)(_make_gather)
_make_scatter = functools.lru_cache(maxsize=None)(_make_scatter)


def _perm_w2(w2, b2, in_dim):
    k = w2.shape[0]
    w2p = w2.reshape(k, in_dim, 8).transpose(0, 2, 1).reshape(k, in_dim * 8)
    b2p = b2.reshape(in_dim, 8).T.reshape(1, in_dim * 8)
    return w2p, b2p


def kernel(x, edge_index, edge_attr, batch,
           en1_w1, en1_b1, en1_w2, en1_b2, root1, bias1,
           en2_w1, en2_b1, en2_w2, en2_b2, root2, bias2,
           mlp_w1, mlp_b1, mlp_w2, mlp_b2):
    pad = _EP - _E
    src_p = jnp.concatenate([edge_index[0], jnp.zeros((pad,), jnp.int32)])
    dst_p = jnp.concatenate([edge_index[1], jnp.full((pad,), _N, jnp.int32)])
    ea_p = jnp.concatenate(
        [edge_attr, jnp.zeros((pad, _ED), jnp.float32)], axis=0)

    w2p1, b2p1 = _perm_w2(en1_w2, en1_b2, _IN)
    w2p2, b2p2 = _perm_w2(en2_w2, en2_b2, _H)
    zeros = jnp.zeros((_NP // 16, _W), jnp.float32)

    xs = _make_gather(_IN)(x, src_p)
    msg1 = _edge_call(ea_p, xs, en1_w1, en1_b1.reshape(1, -1),
                      w2p1, b2p1, block=1024, in_dim=_IN)
    p1 = _make_scatter()(msg1, dst_p, zeros)
    hn = _node_call(x, p1, root1, bias1.reshape(1, -1))
    hs = _make_gather(_IN)(hn, src_p)
    msg2 = _edge_call(ea_p, hs, en2_w1, en2_b1.reshape(1, -1),
                      w2p2, b2p2, block=2048, in_dim=_H)
    p2 = _make_scatter()(msg2, dst_p, zeros)
    return _final_call(hn, p2, batch.reshape(1, -1), root2,
                       bias2.reshape(1, -1), mlp_w1,
                       mlp_b1.reshape(1, -1), mlp_w2, mlp_b2.reshape(1, -1))
